# Initial kernel scaffold; baseline (speedup 1.0000x reference)
import jax
import jax.numpy as jnp
from jax.experimental import pallas as pl


def kernel(sha_herb_edges1, sha_herb_edges2, sha_sym_edges1, sha_sym_edges2, hh_edges, ss_edges, kgOneHot, feature, sids, params):
    raise NotImplementedError("write your pallas kernel here")


# speedup vs baseline: 50.5210x; 50.5210x over previous
import functools

import jax
import jax.numpy as jnp
from jax import lax
from jax.experimental import pallas as pl
from jax.experimental.pallas import tpu as pltpu
from jax.experimental.pallas import tpu_sc as plsc

N_SYM_C = 390
N_HERB_C = 811
N_ATT_C = 35
HID_C = 256
HEADS_C = 8

NHP = 896
NSP = 512
CH = 1024
ROWS_H = NHP // 16
ROWS_S = NSP // 16
LH = ROWS_H * NHP


def _sc_count_body(h1s, h1d, h2s, h2d, hhs, hhd, s1s, s1d, s2s, s2d, sss, ssd,
                   zeros_h, o_h1, o_h2, o_hh, o_s1, o_s2, o_ss,
                   cl, srcv, dstv):
    core = lax.axis_index("c")
    sub = lax.axis_index("s")
    ones = jnp.full((16,), 1.0, jnp.float32)

    def do_list(s_h, d_h, o_h, npad):
        rows = npad // 16
        band = rows * npad
        lo = sub * rows
        nch = s_h.shape[0] // CH
        pltpu.sync_copy(zeros_h.at[pl.ds(0, band)], cl.at[pl.ds(0, band)])

        def chunk(ci, carry):
            pltpu.sync_copy(s_h.at[pl.ds(ci * CH, CH)], srcv)
            pltpu.sync_copy(d_h.at[pl.ds(ci * CH, CH)], dstv)

            def grp(i, c2):
                dd = dstv[pl.ds(i * 16, 16)]
                sv = srcv[pl.ds(i * 16, 16)]
                m = (dd >= lo) & (dd < lo + rows)
                idx = jnp.where(m, (dd - lo) * npad + sv, LH)
                plsc.addupdate_scatter(cl, [idx], ones)
                return c2

            return lax.fori_loop(0, CH // 16, grp, carry, unroll=4)

        lax.fori_loop(0, nch, chunk, 0)
        pltpu.sync_copy(cl.at[pl.ds(0, band)], o_h.at[pl.ds(sub * band, band)])

    @pl.when(core == 0)
    def _core0():
        do_list(h1s, h1d, o_h1, NHP)
        do_list(h2s, h2d, o_h2, NHP)
        do_list(s1s, s1d, o_s1, NSP)

    @pl.when(core == 1)
    def _core1():
        do_list(hhs, hhd, o_hh, NHP)
        do_list(s2s, s2d, o_s2, NSP)
        do_list(sss, ssd, o_ss, NSP)


_sc_counts = functools.partial(
    pl.kernel,
    mesh=plsc.VectorSubcoreMesh(core_axis_name="c", subcore_axis_name="s"),
    compiler_params=pltpu.CompilerParams(needs_layout_passes=False),
    out_type=[
        jax.ShapeDtypeStruct((NHP * NHP,), jnp.float32),
        jax.ShapeDtypeStruct((NHP * NHP,), jnp.float32),
        jax.ShapeDtypeStruct((NHP * NHP,), jnp.float32),
        jax.ShapeDtypeStruct((NSP * NSP,), jnp.float32),
        jax.ShapeDtypeStruct((NSP * NSP,), jnp.float32),
        jax.ShapeDtypeStruct((NSP * NSP,), jnp.float32),
    ],
    scratch_types=[
        pltpu.VMEM((LH + 16,), jnp.float32),
        pltpu.VMEM((CH,), jnp.int32),
        pltpu.VMEM((CH,), jnp.int32),
    ],
)(_sc_count_body)


def _pad_edges(e, ch):
    n = e.shape[1]
    epad = ((n + ch - 1) // ch) * ch
    src = jnp.pad(e[0].astype(jnp.int32), (0, epad - n))
    dst = jnp.pad(e[1].astype(jnp.int32), (0, epad - n), constant_values=-1)
    return src, dst


def _gat_body(x_ref, w_ref, al_ref, ar_ref, c_ref, o_ref):
    x = x_ref[...]
    h = jnp.dot(x, w_ref[...], preferred_element_type=jnp.float32)
    al = al_ref[0]
    ar = ar_ref[0]
    el = lax.dot_general(al, h, (((1,), (1,)), ((), ())),
                         preferred_element_type=jnp.float32)
    er = lax.dot_general(h, ar, (((1,), (1,)), ((), ())),
                         preferred_element_type=jnp.float32)
    e = el + er
    e = jnp.where(e >= 0.0, e, 0.2 * e)
    cmat = c_ref[...]
    mask = cmat > 0.0
    emax = jnp.max(jnp.where(mask, e, -1e30), axis=1, keepdims=True)
    emax = jnp.where(emax > -1e29, emax, 0.0)
    p = jnp.where(mask, jnp.exp(e - emax), 0.0) * cmat
    den = jnp.sum(p, axis=1, keepdims=True)
    alpha = p / (den + 1e-9)
    out = jnp.dot(alpha, h, preferred_element_type=jnp.float32)
    o_ref[...] = jnp.where(out > 0.0, out, jnp.exp(jnp.minimum(out, 0.0)) - 1.0)


def _gat_attention(x, w, al, ar, cmat, npad):
    return pl.pallas_call(
        _gat_body,
        grid=(HEADS_C,),
        in_specs=[
            pl.BlockSpec((npad, HID_C), lambda k: (0, 0)),
            pl.BlockSpec((HID_C, HID_C), lambda k: (0, k)),
            pl.BlockSpec((1, 1, HID_C), lambda k: (k, 0, 0)),
            pl.BlockSpec((1, 1, HID_C), lambda k: (k, 0, 0)),
            pl.BlockSpec((npad, npad), lambda k: (0, 0)),
        ],
        out_specs=pl.BlockSpec((npad, HID_C), lambda k: (0, k)),
        out_shape=jax.ShapeDtypeStruct((npad, HEADS_C * HID_C), jnp.float32),
    )(x, w, al.reshape(HEADS_C, 1, HID_C), ar.reshape(HEADS_C, 1, HID_C), cmat)


def _herbinput_body(h1_ref, kg_ref, att_ref, o_ref):
    kg1 = jnp.dot(kg_ref[...], att_ref[...], preferred_element_type=jnp.float32)
    o_ref[...] = (h1_ref[...] + kg1) * 0.5


def _semantic_body(n, e0_ref, e1_ref, s1w_ref, s1b_ref, s2_ref, pw_ref, pb_ref,
                   o_ref):
    e0 = e0_ref[...]
    e1 = e1_ref[...]
    t0 = jnp.tanh(jnp.dot(e0, s1w_ref[...], preferred_element_type=jnp.float32)
                  + s1b_ref[...])
    t1 = jnp.tanh(jnp.dot(e1, s1w_ref[...], preferred_element_type=jnp.float32)
                  + s1b_ref[...])
    w0 = jnp.sum(jnp.dot(t0, s2_ref[...], preferred_element_type=jnp.float32))
    w1 = jnp.sum(jnp.dot(t1, s2_ref[...], preferred_element_type=jnp.float32))
    w0 = w0 * (1.0 / n)
    w1 = w1 * (1.0 / n)
    m = jnp.maximum(w0, w1)
    b0 = jnp.exp(w0 - m)
    b1 = jnp.exp(w1 - m)
    s = (b0 * e0 + b1 * e1) * (1.0 / (b0 + b1))
    o_ref[...] = (jnp.dot(s, pw_ref[...], preferred_element_type=jnp.float32)
                  + pb_ref[...])


def _semantic(n, e0, e1, s1w, s1b, s2, pw, pb):
    return pl.pallas_call(
        functools.partial(_semantic_body, n),
        out_shape=jax.ShapeDtypeStruct((n, HID_C), jnp.float32),
    )(e0, e1, s1w, s1b.reshape(1, -1), s2, pw, pb.reshape(1, -1))


def _batchnorm_tanh(t):
    m = jnp.mean(t, axis=0, keepdims=True)
    v = jnp.mean((t - m) ** 2, axis=0, keepdims=True)
    return jnp.tanh((t - m) * lax.rsqrt(v + 1e-5))


def _finalize_body(x1_ref, x2_ref, g_ref, w_ref, b_ref, o_ref):
    g = jnp.tanh(g_ref[...])
    acc = g[:, 0:HID_C]
    for k in range(1, HEADS_C):
        acc = acc + g[:, k * HID_C:(k + 1) * HID_C]
    x3 = acc * (1.0 / HEADS_C)
    t = (x1_ref[...] + x2_ref[...] + x3) * (1.0 / 3.0)
    t = jnp.dot(t, w_ref[...], preferred_element_type=jnp.float32) + b_ref[...]
    o_ref[...] = _batchnorm_tanh(t)


def _finalize(x1, x2, g, w, b):
    n = x1.shape[0]
    return pl.pallas_call(
        _finalize_body,
        out_shape=jax.ShapeDtypeStruct((n, HID_C), jnp.float32),
    )(x1, x2, g, w, b.reshape(1, -1))


def _attr_body(a_ref, w_ref, b_ref, o_ref):
    t = (jnp.dot(a_ref[...], w_ref[...], preferred_element_type=jnp.float32)
         + b_ref[...])
    o_ref[...] = _batchnorm_tanh(t)


def kernel(sha_herb_edges1, sha_herb_edges2, sha_sym_edges1, sha_sym_edges2,
           hh_edges, ss_edges, kgOneHot, feature, sids, params):
    del sids
    emb = params['emb']
    allf = jnp.take(emb, feature[:, 0], axis=0)
    symptom1 = allf[:N_SYM_C]
    herb1 = allf[N_SYM_C:N_SYM_C + N_HERB_C]
    attribute1 = allf[N_SYM_C + N_HERB_C:]

    h1s, h1d = _pad_edges(sha_herb_edges1, CH)
    h2s, h2d = _pad_edges(sha_herb_edges2, CH)
    hhs, hhd = _pad_edges(hh_edges, CH)
    s1s, s1d = _pad_edges(sha_sym_edges1, CH)
    s2s, s2d = _pad_edges(sha_sym_edges2, CH)
    sss, ssd = _pad_edges(ss_edges, CH)
    zeros = jnp.zeros((LH,), jnp.float32)

    c_h1, c_h2, c_hh, c_s1, c_s2, c_ss = _sc_counts(
        h1s, h1d, h2s, h2d, hhs, hhd, s1s, s1d, s2s, s2d, sss, ssd, zeros)
    c_h1 = c_h1.reshape(NHP, NHP)
    c_h2 = c_h2.reshape(NHP, NHP)
    c_hh = c_hh.reshape(NHP, NHP)
    c_s1 = c_s1.reshape(NSP, NSP)
    c_s2 = c_s2.reshape(NSP, NSP)
    c_ss = c_ss.reshape(NSP, NSP)

    herbinput = pl.pallas_call(
        _herbinput_body,
        out_shape=jax.ShapeDtypeStruct((N_HERB_C, HID_C), jnp.float32),
    )(herb1, kgOneHot.astype(jnp.float32), attribute1)

    xh = jnp.pad(herb1, ((0, NHP - N_HERB_C), (0, 0)))
    xhi = jnp.pad(herbinput, ((0, NHP - N_HERB_C), (0, 0)))
    xs = jnp.pad(symptom1, ((0, NSP - N_SYM_C), (0, 0)))

    ph = params['H_HAN']
    ps = params['S_HAN']
    g_h1 = _gat_attention(xh, ph['gats'][0]['W'], ph['gats'][0]['al'],
                          ph['gats'][0]['ar'], c_h1, NHP)
    g_h2 = _gat_attention(xh, ph['gats'][1]['W'], ph['gats'][1]['al'],
                          ph['gats'][1]['ar'], c_h2, NHP)
    g_hh = _gat_attention(xhi, params['hh']['W'], params['hh']['al'],
                          params['hh']['ar'], c_hh, NHP)
    g_s1 = _gat_attention(xs, ps['gats'][0]['W'], ps['gats'][0]['al'],
                          ps['gats'][0]['ar'], c_s1, NSP)
    g_s2 = _gat_attention(xs, ps['gats'][1]['W'], ps['gats'][1]['al'],
                          ps['gats'][1]['ar'], c_s2, NSP)
    g_ss = _gat_attention(xs, params['ss']['W'], params['ss']['al'],
                          params['ss']['ar'], c_ss, NSP)

    herb2 = _semantic(N_HERB_C, g_h1[:N_HERB_C], g_h2[:N_HERB_C],
                      ph['sem1']['W'], ph['sem1']['b'], ph['sem2'],
                      ph['pred']['W'], ph['pred']['b'])
    symptom2 = _semantic(N_SYM_C, g_s1[:N_SYM_C], g_s2[:N_SYM_C],
                         ps['sem1']['W'], ps['sem1']['b'], ps['sem2'],
                         ps['pred']['W'], ps['pred']['b'])

    herb = _finalize(herb1, herb2, g_hh[:N_HERB_C],
                     params['H_mlp']['W'], params['H_mlp']['b'])
    symptom = _finalize(symptom1, symptom2, g_ss[:N_SYM_C],
                        params['S_mlp']['W'], params['S_mlp']['b'])
    attribute = pl.pallas_call(
        _attr_body,
        out_shape=jax.ShapeDtypeStruct((N_ATT_C, HID_C), jnp.float32),
    )(attribute1, params['A_mlp']['W'], params['A_mlp']['b'].reshape(1, -1))

    return herb, symptom, attribute


# speedup vs baseline: 109.9076x; 2.1755x over previous
import functools

import jax
import jax.numpy as jnp
from jax import lax
from jax.experimental import pallas as pl
from jax.experimental.pallas import tpu as pltpu
from jax.experimental.pallas import tpu_sc as plsc

N_SYM_C = 390
N_HERB_C = 811
N_ATT_C = 35
HID_C = 256
HEADS_C = 8

NHP = 896
NSP = 512
EPH = 53248
EPS = 28672
SH_H = EPH // 16
SEG_H = NHP * NHP // 16


def _sc_count_body(h1s, h1d, h2s, h2d, hhs, hhd, s1s, s1d, s2s, s2d, sss, ssd,
                   zeros_h, ones_h, o_h1, o_h2, o_hh, o_s1, o_s2, o_ss,
                   shmat, srcv, dstv, idxv, onesv):
    core = lax.axis_index("c")
    sub = lax.axis_index("s")
    pltpu.sync_copy(ones_h, onesv)

    def do_list(s_h, d_h, o_h, npad):
        share = s_h.shape[0] // 16
        seg = npad * npad // 16
        plsc.subcore_barrier()
        pltpu.sync_copy(zeros_h.at[pl.ds(sub * seg, seg)],
                        shmat.at[pl.ds(sub * seg, seg)])
        pltpu.sync_copy(s_h.at[pl.ds(sub * share, share)],
                        srcv.at[pl.ds(0, share)])
        pltpu.sync_copy(d_h.at[pl.ds(sub * share, share)],
                        dstv.at[pl.ds(0, share)])

        def grp(i, c2):
            idxv[pl.ds(i * 16, 16)] = (dstv[pl.ds(i * 16, 16)] * npad
                                       + srcv[pl.ds(i * 16, 16)])
            return c2

        lax.fori_loop(0, share // 16, grp, 0, unroll=8)
        plsc.subcore_barrier()
        pltpu.sync_copy(onesv.at[pl.ds(0, share)],
                        shmat.at[idxv.at[pl.ds(0, share)]], add=True)
        plsc.subcore_barrier()
        pltpu.sync_copy(shmat.at[pl.ds(sub * seg, seg)],
                        o_h.at[pl.ds(sub * seg, seg)])

    @pl.when(core == 0)
    def _core0():
        do_list(h1s, h1d, o_h1, NHP)
        do_list(h2s, h2d, o_h2, NHP)
        do_list(s1s, s1d, o_s1, NSP)

    @pl.when(core == 1)
    def _core1():
        do_list(hhs, hhd, o_hh, NHP)
        do_list(s2s, s2d, o_s2, NSP)
        do_list(sss, ssd, o_ss, NSP)


_sc_counts = functools.partial(
    pl.kernel,
    mesh=plsc.VectorSubcoreMesh(core_axis_name="c", subcore_axis_name="s"),
    compiler_params=pltpu.CompilerParams(needs_layout_passes=False),
    out_type=[
        jax.ShapeDtypeStruct((NHP * NHP,), jnp.float32),
        jax.ShapeDtypeStruct((NHP * NHP,), jnp.float32),
        jax.ShapeDtypeStruct((NHP * NHP,), jnp.float32),
        jax.ShapeDtypeStruct((NSP * NSP,), jnp.float32),
        jax.ShapeDtypeStruct((NSP * NSP,), jnp.float32),
        jax.ShapeDtypeStruct((NSP * NSP,), jnp.float32),
    ],
    scratch_types=[
        pltpu.VMEM_SHARED((NHP * NHP,), jnp.float32),
        pltpu.VMEM((SH_H,), jnp.int32),
        pltpu.VMEM((SH_H,), jnp.int32),
        pltpu.VMEM((SH_H,), jnp.int32),
        pltpu.VMEM((SH_H,), jnp.float32),
    ],
)(_sc_count_body)


def _pad_edges(e, epad, npad):
    n = e.shape[1]
    src = jnp.pad(e[0].astype(jnp.int32), (0, epad - n),
                  constant_values=npad - 1)
    dst = jnp.pad(e[1].astype(jnp.int32), (0, epad - n),
                  constant_values=npad - 1)
    return src, dst


def _gat_body(x_ref, w_ref, al_ref, ar_ref, c_ref, o_ref):
    x = x_ref[...]
    h = jnp.dot(x, w_ref[...], preferred_element_type=jnp.float32)
    al = al_ref[0]
    ar = ar_ref[0]
    el = lax.dot_general(al, h, (((1,), (1,)), ((), ())),
                         preferred_element_type=jnp.float32)
    er = lax.dot_general(h, ar, (((1,), (1,)), ((), ())),
                         preferred_element_type=jnp.float32)
    e = el + er
    e = jnp.where(e >= 0.0, e, 0.2 * e)
    cmat = c_ref[...]
    mask = cmat > 0.0
    emax = jnp.max(jnp.where(mask, e, -1e30), axis=1, keepdims=True)
    emax = jnp.where(emax > -1e29, emax, 0.0)
    p = jnp.where(mask, jnp.exp(e - emax), 0.0) * cmat
    den = jnp.sum(p, axis=1, keepdims=True)
    alpha = p / (den + 1e-9)
    out = jnp.dot(alpha, h, preferred_element_type=jnp.float32)
    o_ref[...] = jnp.where(out > 0.0, out, jnp.exp(jnp.minimum(out, 0.0)) - 1.0)


def _gat_attention(x, w, al, ar, cmat, npad):
    return pl.pallas_call(
        _gat_body,
        grid=(HEADS_C,),
        in_specs=[
            pl.BlockSpec((npad, HID_C), lambda k: (0, 0)),
            pl.BlockSpec((HID_C, HID_C), lambda k: (0, k)),
            pl.BlockSpec((1, 1, HID_C), lambda k: (k, 0, 0)),
            pl.BlockSpec((1, 1, HID_C), lambda k: (k, 0, 0)),
            pl.BlockSpec((npad, npad), lambda k: (0, 0)),
        ],
        out_specs=pl.BlockSpec((npad, HID_C), lambda k: (0, k)),
        out_shape=jax.ShapeDtypeStruct((npad, HEADS_C * HID_C), jnp.float32),
    )(x, w, al.reshape(HEADS_C, 1, HID_C), ar.reshape(HEADS_C, 1, HID_C), cmat)


def _herbinput_body(h1_ref, kg_ref, att_ref, o_ref):
    kg1 = jnp.dot(kg_ref[...], att_ref[...], preferred_element_type=jnp.float32)
    o_ref[...] = (h1_ref[...] + kg1) * 0.5


def _semantic_body(n, e0_ref, e1_ref, s1w_ref, s1b_ref, s2_ref, pw_ref, pb_ref,
                   o_ref):
    e0 = e0_ref[...]
    e1 = e1_ref[...]
    t0 = jnp.tanh(jnp.dot(e0, s1w_ref[...], preferred_element_type=jnp.float32)
                  + s1b_ref[...])
    t1 = jnp.tanh(jnp.dot(e1, s1w_ref[...], preferred_element_type=jnp.float32)
                  + s1b_ref[...])
    w0 = jnp.sum(jnp.dot(t0, s2_ref[...], preferred_element_type=jnp.float32))
    w1 = jnp.sum(jnp.dot(t1, s2_ref[...], preferred_element_type=jnp.float32))
    w0 = w0 * (1.0 / n)
    w1 = w1 * (1.0 / n)
    m = jnp.maximum(w0, w1)
    b0 = jnp.exp(w0 - m)
    b1 = jnp.exp(w1 - m)
    s = (b0 * e0 + b1 * e1) * (1.0 / (b0 + b1))
    o_ref[...] = (jnp.dot(s, pw_ref[...], preferred_element_type=jnp.float32)
                  + pb_ref[...])


def _semantic(n, e0, e1, s1w, s1b, s2, pw, pb):
    return pl.pallas_call(
        functools.partial(_semantic_body, n),
        out_shape=jax.ShapeDtypeStruct((n, HID_C), jnp.float32),
    )(e0, e1, s1w, s1b.reshape(1, -1), s2, pw, pb.reshape(1, -1))


def _batchnorm_tanh(t):
    m = jnp.mean(t, axis=0, keepdims=True)
    v = jnp.mean((t - m) ** 2, axis=0, keepdims=True)
    return jnp.tanh((t - m) * lax.rsqrt(v + 1e-5))


def _finalize_body(x1_ref, x2_ref, g_ref, w_ref, b_ref, o_ref):
    g = jnp.tanh(g_ref[...])
    acc = g[:, 0:HID_C]
    for k in range(1, HEADS_C):
        acc = acc + g[:, k * HID_C:(k + 1) * HID_C]
    x3 = acc * (1.0 / HEADS_C)
    t = (x1_ref[...] + x2_ref[...] + x3) * (1.0 / 3.0)
    t = jnp.dot(t, w_ref[...], preferred_element_type=jnp.float32) + b_ref[...]
    o_ref[...] = _batchnorm_tanh(t)


def _finalize(x1, x2, g, w, b):
    n = x1.shape[0]
    return pl.pallas_call(
        _finalize_body,
        out_shape=jax.ShapeDtypeStruct((n, HID_C), jnp.float32),
    )(x1, x2, g, w, b.reshape(1, -1))


def _attr_body(a_ref, w_ref, b_ref, o_ref):
    t = (jnp.dot(a_ref[...], w_ref[...], preferred_element_type=jnp.float32)
         + b_ref[...])
    o_ref[...] = _batchnorm_tanh(t)


def kernel(sha_herb_edges1, sha_herb_edges2, sha_sym_edges1, sha_sym_edges2,
           hh_edges, ss_edges, kgOneHot, feature, sids, params):
    del sids
    emb = params['emb']
    allf = jnp.take(emb, feature[:, 0], axis=0)
    symptom1 = allf[:N_SYM_C]
    herb1 = allf[N_SYM_C:N_SYM_C + N_HERB_C]
    attribute1 = allf[N_SYM_C + N_HERB_C:]

    h1s, h1d = _pad_edges(sha_herb_edges1, EPH, NHP)
    h2s, h2d = _pad_edges(sha_herb_edges2, EPH, NHP)
    hhs, hhd = _pad_edges(hh_edges, EPH, NHP)
    s1s, s1d = _pad_edges(sha_sym_edges1, EPS, NSP)
    s2s, s2d = _pad_edges(sha_sym_edges2, EPS, NSP)
    sss, ssd = _pad_edges(ss_edges, EPS, NSP)
    zeros = jnp.zeros((NHP * NHP,), jnp.float32)
    ones = jnp.ones((SH_H,), jnp.float32)

    c_h1, c_h2, c_hh, c_s1, c_s2, c_ss = _sc_counts(
        h1s, h1d, h2s, h2d, hhs, hhd, s1s, s1d, s2s, s2d, sss, ssd,
        zeros, ones)
    c_h1 = c_h1.reshape(NHP, NHP)
    c_h2 = c_h2.reshape(NHP, NHP)
    c_hh = c_hh.reshape(NHP, NHP)
    c_s1 = c_s1.reshape(NSP, NSP)
    c_s2 = c_s2.reshape(NSP, NSP)
    c_ss = c_ss.reshape(NSP, NSP)

    herbinput = pl.pallas_call(
        _herbinput_body,
        out_shape=jax.ShapeDtypeStruct((N_HERB_C, HID_C), jnp.float32),
    )(herb1, kgOneHot.astype(jnp.float32), attribute1)

    xh = jnp.pad(herb1, ((0, NHP - N_HERB_C), (0, 0)))
    xhi = jnp.pad(herbinput, ((0, NHP - N_HERB_C), (0, 0)))
    xs = jnp.pad(symptom1, ((0, NSP - N_SYM_C), (0, 0)))

    ph = params['H_HAN']
    ps = params['S_HAN']
    g_h1 = _gat_attention(xh, ph['gats'][0]['W'], ph['gats'][0]['al'],
                          ph['gats'][0]['ar'], c_h1, NHP)
    g_h2 = _gat_attention(xh, ph['gats'][1]['W'], ph['gats'][1]['al'],
                          ph['gats'][1]['ar'], c_h2, NHP)
    g_hh = _gat_attention(xhi, params['hh']['W'], params['hh']['al'],
                          params['hh']['ar'], c_hh, NHP)
    g_s1 = _gat_attention(xs, ps['gats'][0]['W'], ps['gats'][0]['al'],
                          ps['gats'][0]['ar'], c_s1, NSP)
    g_s2 = _gat_attention(xs, ps['gats'][1]['W'], ps['gats'][1]['al'],
                          ps['gats'][1]['ar'], c_s2, NSP)
    g_ss = _gat_attention(xs, params['ss']['W'], params['ss']['al'],
                          params['ss']['ar'], c_ss, NSP)

    herb2 = _semantic(N_HERB_C, g_h1[:N_HERB_C], g_h2[:N_HERB_C],
                      ph['sem1']['W'], ph['sem1']['b'], ph['sem2'],
                      ph['pred']['W'], ph['pred']['b'])
    symptom2 = _semantic(N_SYM_C, g_s1[:N_SYM_C], g_s2[:N_SYM_C],
                         ps['sem1']['W'], ps['sem1']['b'], ps['sem2'],
                         ps['pred']['W'], ps['pred']['b'])

    herb = _finalize(herb1, herb2, g_hh[:N_HERB_C],
                     params['H_mlp']['W'], params['H_mlp']['b'])
    symptom = _finalize(symptom1, symptom2, g_ss[:N_SYM_C],
                        params['S_mlp']['W'], params['S_mlp']['b'])
    attribute = pl.pallas_call(
        _attr_body,
        out_shape=jax.ShapeDtypeStruct((N_ATT_C, HID_C), jnp.float32),
    )(attribute1, params['A_mlp']['W'], params['A_mlp']['b'].reshape(1, -1))

    return herb, symptom, attribute


# speedup vs baseline: 112.8700x; 1.0270x over previous
import functools

import jax
import jax.numpy as jnp
from jax import lax
from jax.experimental import pallas as pl
from jax.experimental.pallas import tpu as pltpu
from jax.experimental.pallas import tpu_sc as plsc

N_SYM_C = 390
N_HERB_C = 811
N_ATT_C = 35
HID_C = 256
HEADS_C = 8

NHP = 896
NSP = 512
EPH = 53248
EPS = 28672
SH_H = EPH // 16
SEG_H = NHP * NHP // 16
ZCH = SEG_H // 8


def _sc_count_body(h1s, h1d, h2s, h2d, hhs, hhd, s1s, s1d, s2s, s2d, sss, ssd,
                   ones_h, o_h1, o_h2, o_hh, o_s1, o_s2, o_ss,
                   shmat, srcv, dstv, idxv, onesv, zbuf):
    core = lax.axis_index("c")
    sub = lax.axis_index("s")
    pltpu.sync_copy(ones_h, onesv)
    zv = jnp.zeros((16,), jnp.float32)

    def zfill(i, c2):
        zbuf[pl.ds(i * 16, 16)] = zv
        return c2

    lax.fori_loop(0, ZCH // 16, zfill, 0, unroll=8)

    def do_list(s_h, d_h, o_h, npad):
        share = s_h.shape[0] // 16
        seg = npad * npad // 16
        plsc.subcore_barrier()
        base = sub * seg
        cs = seg // 8
        for j in range(8):
            pltpu.sync_copy(zbuf.at[pl.ds(0, cs)],
                            shmat.at[pl.ds(base + j * cs, cs)])
        pltpu.sync_copy(s_h.at[pl.ds(sub * share, share)],
                        srcv.at[pl.ds(0, share)])
        pltpu.sync_copy(d_h.at[pl.ds(sub * share, share)],
                        dstv.at[pl.ds(0, share)])

        def grp(i, c2):
            idxv[pl.ds(i * 16, 16)] = (dstv[pl.ds(i * 16, 16)] * npad
                                       + srcv[pl.ds(i * 16, 16)])
            return c2

        lax.fori_loop(0, share // 16, grp, 0, unroll=8)
        plsc.subcore_barrier()
        pltpu.sync_copy(onesv.at[pl.ds(0, share)],
                        shmat.at[idxv.at[pl.ds(0, share)]], add=True)
        plsc.subcore_barrier()
        pltpu.sync_copy(shmat.at[pl.ds(sub * seg, seg)],
                        o_h.at[pl.ds(sub * seg, seg)])

    @pl.when(core == 0)
    def _core0():
        do_list(h1s, h1d, o_h1, NHP)
        do_list(h2s, h2d, o_h2, NHP)
        do_list(s1s, s1d, o_s1, NSP)

    @pl.when(core == 1)
    def _core1():
        do_list(hhs, hhd, o_hh, NHP)
        do_list(s2s, s2d, o_s2, NSP)
        do_list(sss, ssd, o_ss, NSP)


_sc_counts = functools.partial(
    pl.kernel,
    mesh=plsc.VectorSubcoreMesh(core_axis_name="c", subcore_axis_name="s"),
    compiler_params=pltpu.CompilerParams(needs_layout_passes=False),
    out_type=[
        jax.ShapeDtypeStruct((NHP * NHP,), jnp.float32),
        jax.ShapeDtypeStruct((NHP * NHP,), jnp.float32),
        jax.ShapeDtypeStruct((NHP * NHP,), jnp.float32),
        jax.ShapeDtypeStruct((NSP * NSP,), jnp.float32),
        jax.ShapeDtypeStruct((NSP * NSP,), jnp.float32),
        jax.ShapeDtypeStruct((NSP * NSP,), jnp.float32),
    ],
    scratch_types=[
        pltpu.VMEM_SHARED((NHP * NHP,), jnp.float32),
        pltpu.VMEM((SH_H,), jnp.int32),
        pltpu.VMEM((SH_H,), jnp.int32),
        pltpu.VMEM((SH_H,), jnp.int32),
        pltpu.VMEM((SH_H,), jnp.float32),
        pltpu.VMEM((ZCH,), jnp.float32),
    ],
)(_sc_count_body)


def _pad_edges(e, epad, npad):
    n = e.shape[1]
    src = jnp.pad(e[0].astype(jnp.int32), (0, epad - n),
                  constant_values=npad - 1)
    dst = jnp.pad(e[1].astype(jnp.int32), (0, epad - n),
                  constant_values=npad - 1)
    return src, dst


def _gat_body(x_ref, w_ref, al_ref, ar_ref, c_ref, o_ref):
    x = x_ref[...]
    h = jnp.dot(x, w_ref[...], preferred_element_type=jnp.float32)
    al = al_ref[0]
    ar = ar_ref[0]
    el = lax.dot_general(al, h, (((1,), (1,)), ((), ())),
                         preferred_element_type=jnp.float32)
    er = lax.dot_general(h, ar, (((1,), (1,)), ((), ())),
                         preferred_element_type=jnp.float32)
    e = el + er
    e = jnp.where(e >= 0.0, e, 0.2 * e)
    cmat = c_ref[...]
    mask = cmat > 0.0
    emax = jnp.max(jnp.where(mask, e, -1e30), axis=1, keepdims=True)
    emax = jnp.where(emax > -1e29, emax, 0.0)
    p = jnp.where(mask, jnp.exp(e - emax), 0.0) * cmat
    den = jnp.sum(p, axis=1, keepdims=True)
    alpha = p / (den + 1e-9)
    out = jnp.dot(alpha, h, preferred_element_type=jnp.float32)
    o_ref[...] = jnp.where(out > 0.0, out, jnp.exp(jnp.minimum(out, 0.0)) - 1.0)


def _gat_attention(x, w, al, ar, cmat, npad):
    return pl.pallas_call(
        _gat_body,
        grid=(HEADS_C,),
        in_specs=[
            pl.BlockSpec((npad, HID_C), lambda k: (0, 0)),
            pl.BlockSpec((HID_C, HID_C), lambda k: (0, k)),
            pl.BlockSpec((1, 1, HID_C), lambda k: (k, 0, 0)),
            pl.BlockSpec((1, 1, HID_C), lambda k: (k, 0, 0)),
            pl.BlockSpec((npad, npad), lambda k: (0, 0)),
        ],
        out_specs=pl.BlockSpec((npad, HID_C), lambda k: (0, k)),
        out_shape=jax.ShapeDtypeStruct((npad, HEADS_C * HID_C), jnp.float32),
    )(x, w, al.reshape(HEADS_C, 1, HID_C), ar.reshape(HEADS_C, 1, HID_C), cmat)


def _herbinput_body(h1_ref, kg_ref, att_ref, o_ref):
    kg1 = jnp.dot(kg_ref[...], att_ref[...], preferred_element_type=jnp.float32)
    o_ref[...] = (h1_ref[...] + kg1) * 0.5


def _semantic_body(n, e0_ref, e1_ref, s1w_ref, s1b_ref, s2_ref, pw_ref, pb_ref,
                   o_ref):
    e0 = e0_ref[...]
    e1 = e1_ref[...]
    t0 = jnp.tanh(jnp.dot(e0, s1w_ref[...], preferred_element_type=jnp.float32)
                  + s1b_ref[...])
    t1 = jnp.tanh(jnp.dot(e1, s1w_ref[...], preferred_element_type=jnp.float32)
                  + s1b_ref[...])
    w0 = jnp.sum(jnp.dot(t0, s2_ref[...], preferred_element_type=jnp.float32))
    w1 = jnp.sum(jnp.dot(t1, s2_ref[...], preferred_element_type=jnp.float32))
    w0 = w0 * (1.0 / n)
    w1 = w1 * (1.0 / n)
    m = jnp.maximum(w0, w1)
    b0 = jnp.exp(w0 - m)
    b1 = jnp.exp(w1 - m)
    s = (b0 * e0 + b1 * e1) * (1.0 / (b0 + b1))
    o_ref[...] = (jnp.dot(s, pw_ref[...], preferred_element_type=jnp.float32)
                  + pb_ref[...])


def _semantic(n, e0, e1, s1w, s1b, s2, pw, pb):
    return pl.pallas_call(
        functools.partial(_semantic_body, n),
        out_shape=jax.ShapeDtypeStruct((n, HID_C), jnp.float32),
    )(e0, e1, s1w, s1b.reshape(1, -1), s2, pw, pb.reshape(1, -1))


def _batchnorm_tanh(t):
    m = jnp.mean(t, axis=0, keepdims=True)
    v = jnp.mean((t - m) ** 2, axis=0, keepdims=True)
    return jnp.tanh((t - m) * lax.rsqrt(v + 1e-5))


def _finalize_body(x1_ref, x2_ref, g_ref, w_ref, b_ref, o_ref):
    g = jnp.tanh(g_ref[...])
    acc = g[:, 0:HID_C]
    for k in range(1, HEADS_C):
        acc = acc + g[:, k * HID_C:(k + 1) * HID_C]
    x3 = acc * (1.0 / HEADS_C)
    t = (x1_ref[...] + x2_ref[...] + x3) * (1.0 / 3.0)
    t = jnp.dot(t, w_ref[...], preferred_element_type=jnp.float32) + b_ref[...]
    o_ref[...] = _batchnorm_tanh(t)


def _finalize(x1, x2, g, w, b):
    n = x1.shape[0]
    return pl.pallas_call(
        _finalize_body,
        out_shape=jax.ShapeDtypeStruct((n, HID_C), jnp.float32),
    )(x1, x2, g, w, b.reshape(1, -1))


def _attr_body(a_ref, w_ref, b_ref, o_ref):
    t = (jnp.dot(a_ref[...], w_ref[...], preferred_element_type=jnp.float32)
         + b_ref[...])
    o_ref[...] = _batchnorm_tanh(t)


def kernel(sha_herb_edges1, sha_herb_edges2, sha_sym_edges1, sha_sym_edges2,
           hh_edges, ss_edges, kgOneHot, feature, sids, params):
    del sids
    emb = params['emb']
    allf = jnp.take(emb, feature[:, 0], axis=0)
    symptom1 = allf[:N_SYM_C]
    herb1 = allf[N_SYM_C:N_SYM_C + N_HERB_C]
    attribute1 = allf[N_SYM_C + N_HERB_C:]

    h1s, h1d = _pad_edges(sha_herb_edges1, EPH, NHP)
    h2s, h2d = _pad_edges(sha_herb_edges2, EPH, NHP)
    hhs, hhd = _pad_edges(hh_edges, EPH, NHP)
    s1s, s1d = _pad_edges(sha_sym_edges1, EPS, NSP)
    s2s, s2d = _pad_edges(sha_sym_edges2, EPS, NSP)
    sss, ssd = _pad_edges(ss_edges, EPS, NSP)
    ones = jnp.ones((SH_H,), jnp.float32)

    c_h1, c_h2, c_hh, c_s1, c_s2, c_ss = _sc_counts(
        h1s, h1d, h2s, h2d, hhs, hhd, s1s, s1d, s2s, s2d, sss, ssd, ones)
    c_h1 = c_h1.reshape(NHP, NHP)
    c_h2 = c_h2.reshape(NHP, NHP)
    c_hh = c_hh.reshape(NHP, NHP)
    c_s1 = c_s1.reshape(NSP, NSP)
    c_s2 = c_s2.reshape(NSP, NSP)
    c_ss = c_ss.reshape(NSP, NSP)

    herbinput = pl.pallas_call(
        _herbinput_body,
        out_shape=jax.ShapeDtypeStruct((N_HERB_C, HID_C), jnp.float32),
    )(herb1, kgOneHot.astype(jnp.float32), attribute1)

    xh = jnp.pad(herb1, ((0, NHP - N_HERB_C), (0, 0)))
    xhi = jnp.pad(herbinput, ((0, NHP - N_HERB_C), (0, 0)))
    xs = jnp.pad(symptom1, ((0, NSP - N_SYM_C), (0, 0)))

    ph = params['H_HAN']
    ps = params['S_HAN']
    g_h1 = _gat_attention(xh, ph['gats'][0]['W'], ph['gats'][0]['al'],
                          ph['gats'][0]['ar'], c_h1, NHP)
    g_h2 = _gat_attention(xh, ph['gats'][1]['W'], ph['gats'][1]['al'],
                          ph['gats'][1]['ar'], c_h2, NHP)
    g_hh = _gat_attention(xhi, params['hh']['W'], params['hh']['al'],
                          params['hh']['ar'], c_hh, NHP)
    g_s1 = _gat_attention(xs, ps['gats'][0]['W'], ps['gats'][0]['al'],
                          ps['gats'][0]['ar'], c_s1, NSP)
    g_s2 = _gat_attention(xs, ps['gats'][1]['W'], ps['gats'][1]['al'],
                          ps['gats'][1]['ar'], c_s2, NSP)
    g_ss = _gat_attention(xs, params['ss']['W'], params['ss']['al'],
                          params['ss']['ar'], c_ss, NSP)

    herb2 = _semantic(N_HERB_C, g_h1[:N_HERB_C], g_h2[:N_HERB_C],
                      ph['sem1']['W'], ph['sem1']['b'], ph['sem2'],
                      ph['pred']['W'], ph['pred']['b'])
    symptom2 = _semantic(N_SYM_C, g_s1[:N_SYM_C], g_s2[:N_SYM_C],
                         ps['sem1']['W'], ps['sem1']['b'], ps['sem2'],
                         ps['pred']['W'], ps['pred']['b'])

    herb = _finalize(herb1, herb2, g_hh[:N_HERB_C],
                     params['H_mlp']['W'], params['H_mlp']['b'])
    symptom = _finalize(symptom1, symptom2, g_ss[:N_SYM_C],
                        params['S_mlp']['W'], params['S_mlp']['b'])
    attribute = pl.pallas_call(
        _attr_body,
        out_shape=jax.ShapeDtypeStruct((N_ATT_C, HID_C), jnp.float32),
    )(attribute1, params['A_mlp']['W'], params['A_mlp']['b'].reshape(1, -1))

    return herb, symptom, attribute
